# baseline (speedup 1.0000x reference)
import functools

import jax
import jax.numpy as jnp
from jax import lax
from jax.experimental import pallas as pl
from jax.experimental.pallas import tpu as pltpu
from jax.experimental.pallas import tpu_sc as plsc

VOCAB = 1000000
EMB = 32
OUT = 5
B = 4096
L = 200

NC = 2
NS = 16
NW = NC * NS
BAGS_PER_W = B // NW
CHUNK_BAGS = 2
TOK_PER_CHUNK = CHUNK_BAGS * L
STREAM_W = 80
STREAMS_PER_CHUNK = TOK_PER_CHUNK // STREAM_W
CHUNKS = BAGS_PER_W // CHUNK_BAGS
PAIR_TOK = 2 * TOK_PER_CHUNK
PAIRS = CHUNKS // 2
TOK_PER_W = BAGS_PER_W * L
W128 = 4 * EMB


def _sc_pool(table4, data1d):
    mesh = plsc.VectorSubcoreMesh(core_axis_name="c", subcore_axis_name="s")

    @functools.partial(
        pl.kernel,
        mesh=mesh,
        out_type=jax.ShapeDtypeStruct((B // 4, W128), jnp.float32),
        scratch_types=[
            pltpu.VMEM((PAIR_TOK + 16,), jnp.int32),
            pltpu.VMEM((PAIR_TOK,), jnp.int32),
            pltpu.VMEM((TOK_PER_CHUNK, W128), jnp.float32),
            pltpu.VMEM((TOK_PER_CHUNK, W128), jnp.float32),
            pltpu.VMEM((1, W128), jnp.float32),
            pltpu.SemaphoreType.DMA,
            pltpu.SemaphoreType.DMA,
            pltpu.SemaphoreType.DMA,
        ],
    )
    def pool(table_hbm, data_hbm, out_hbm, idx_v, row4_v, rows_a, rows_b,
             stage_v, isem, sem_a, sem_b):
        wid = lax.axis_index("s") * NC + lax.axis_index("c")
        tok0 = wid * TOK_PER_W

        def load_pair(k):
            pltpu.async_copy(
                data_hbm.at[pl.ds(tok0 + k * PAIR_TOK, PAIR_TOK)],
                idx_v.at[pl.ds(0, PAIR_TOK)],
                isem,
            ).wait()
            for g in range(PAIR_TOK // 16):
                row4_v[pl.ds(g * 16, 16)] = (
                    lax.shift_right_logical(idx_v[pl.ds(g * 16, 16)], 2)
                )

        def fire(half, rows_v, sem):
            for j in range(STREAMS_PER_CHUNK):
                pltpu.async_copy(
                    table_hbm.at[
                        row4_v.at[pl.ds((half * STREAMS_PER_CHUNK + j) * STREAM_W,
                                        STREAM_W)]
                    ],
                    rows_v.at[pl.ds(j * STREAM_W, STREAM_W)],
                    sem,
                )

        def drain(rows_v, sem):
            pltpu.make_async_copy(
                table_hbm.at[pl.ds(0, TOK_PER_CHUNK)], rows_v, sem
            ).wait()

        def acc(half, rows_v):
            for i in range(CHUNK_BAGS):
                lane0 = (half * CHUNK_BAGS + i) * 32

                def row_body(r, accs):
                    a0, a1 = accs
                    tok = i * L + r * 8
                    svec = (idx_v[pl.ds(half * TOK_PER_CHUNK + tok, 16)] & 3) * 32
                    for u in range(8):
                        s = svec[u]
                        a0 = a0 + rows_v[tok + u, pl.ds(s, 16)]
                        a1 = a1 + rows_v[tok + u, pl.ds(s + 16, 16)]
                    return (a0, a1)

                zero = jnp.zeros((16,), jnp.float32)
                a0, a1 = lax.fori_loop(0, L // 8, row_body, (zero, zero))
                stage_v[0, pl.ds(lane0, 16)] = a0
                stage_v[0, pl.ds(lane0 + 16, 16)] = a1

        def pair_body(k, carry):
            load_pair(k)
            fire(0, rows_a, sem_a)
            fire(1, rows_b, sem_b)
            drain(rows_a, sem_a)
            acc(0, rows_a)
            drain(rows_b, sem_b)
            acc(1, rows_b)
            pltpu.sync_copy(stage_v, out_hbm.at[pl.ds(wid * PAIRS + k, 1)])
            return carry

        lax.fori_loop(0, PAIRS, pair_body, 0)

    return pool(table4, data1d)


def _tc_head(pooled, inv_len, wt, b2):
    BLK = 512

    def body(p_ref, il_ref, w_ref, b_ref, o_ref):
        x = p_ref[:] * il_ref[:]
        y = jnp.dot(x, w_ref[:], preferred_element_type=jnp.float32)
        o_ref[:] = y + b_ref[:]

    return pl.pallas_call(
        body,
        grid=(B // BLK,),
        in_specs=[
            pl.BlockSpec((BLK, EMB), lambda i: (i, 0)),
            pl.BlockSpec((BLK, 1), lambda i: (i, 0)),
            pl.BlockSpec((EMB, OUT), lambda i: (0, 0)),
            pl.BlockSpec((1, OUT), lambda i: (0, 0)),
        ],
        out_specs=pl.BlockSpec((BLK, OUT), lambda i: (i, 0)),
        out_shape=jax.ShapeDtypeStruct((B, OUT), jnp.float32),
    )(pooled, inv_len, wt, b2)


def kernel(data, length, embed, W, b):
    pooled4 = _sc_pool(embed.reshape(VOCAB // 4, W128), data.reshape(B * L))
    pooled = pooled4.reshape(B, EMB)
    inv_len = (1.0 / length.astype(jnp.float32)).reshape(B, 1)
    return _tc_head(pooled, inv_len, W.T, b.reshape(1, OUT))
